# baseline (speedup 1.0000x reference)
import functools

import jax
import jax.numpy as jnp
from jax import lax
from jax.experimental import pallas as pl
from jax.experimental.pallas import tpu as pltpu
from jax.experimental.pallas import tpu_sc as plsc

B = 16384
R_IN = 7
R_OUT = 6
D = 128
LANES = 16

_info = plsc.get_sparse_core_info()
NC, NS = _info.num_cores, _info.num_subcores
NW = NC * NS
PER_W = B // NW
NCHUNK = 16
NSLOT = 4
NSTEPS = PER_W // NCHUNK


def _body(xt_hbm, out_hbm, in_bufs, out_bufs, in_sems, out_sems):
    wid = lax.axis_index("s") * NC + lax.axis_index("c")
    base = wid * PER_W

    def start_in(step, p):
        off = base + step * NCHUNK
        return pltpu.async_copy(
            xt_hbm.at[pl.ds(0, R_IN), pl.ds(off, NCHUNK)],
            in_bufs[p], in_sems[p])

    def wait_in(p):
        pltpu.make_async_copy(
            xt_hbm.at[pl.ds(0, R_IN), pl.ds(0, NCHUNK)],
            in_bufs[p], in_sems[p]).wait()

    def start_out(step, p):
        off = base + step * NCHUNK
        return pltpu.async_copy(
            out_bufs[p], out_hbm.at[pl.ds(off, NCHUNK)], out_sems[p])

    def wait_out(p):
        pltpu.make_async_copy(
            out_bufs[p], out_hbm.at[pl.ds(0, NCHUNK)], out_sems[p]).wait()

    def run_compute(p):
        in_buf, out_buf = in_bufs[p], out_bufs[p]

        @plsc.parallel_loop(0, NCHUNK, unroll=4)
        def compute_one(i):
            for k in range(R_OUT):
                for v in range(D // LANES):
                    sl = pl.ds(v * LANES, LANES)
                    out_buf[i, k // 2, k % 2, sl] = (
                        in_buf[k, i, sl] + in_buf[k + 1, i, sl])

    for p in range(NSLOT - 1):
        start_in(p, p)

    def quad_body(q, carry):
        for p in range(NSLOT):
            s = NSLOT * q + p
            wait_in(p)

            @pl.when(q > 0)
            def _(p=p):
                wait_out(p)
            run_compute(p)
            start_out(s, p)

            @pl.when(s + NSLOT - 1 < NSTEPS)
            def _(s=s, p=p):
                start_in(s + NSLOT - 1, (p + NSLOT - 1) % NSLOT)
        return carry

    lax.fori_loop(0, NSTEPS // NSLOT, quad_body, 0)
    for p in range(NSLOT):
        wait_out(p)


def kernel(x):
    xt = jnp.transpose(x, (1, 0, 2))
    mesh = plsc.VectorSubcoreMesh(core_axis_name="c", subcore_axis_name="s")
    run = functools.partial(
        pl.kernel,
        mesh=mesh,
        out_type=jax.ShapeDtypeStruct((B, 3, 2, D), jnp.float32),
        compiler_params=pltpu.CompilerParams(use_tc_tiling_on_sc=True),
        scratch_types=[
            [pltpu.VMEM((R_IN, NCHUNK, D), jnp.float32) for _ in range(NSLOT)],
            [pltpu.VMEM((NCHUNK, 3, 2, D), jnp.float32) for _ in range(NSLOT)],
            [pltpu.SemaphoreType.DMA for _ in range(NSLOT)],
            [pltpu.SemaphoreType.DMA for _ in range(NSLOT)],
        ],
    )(_body)
    return run(xt)


# speedup vs baseline: 1.0683x; 1.0683x over previous
import functools

import jax
import jax.numpy as jnp
from jax import lax
from jax.experimental import pallas as pl
from jax.experimental.pallas import tpu as pltpu
from jax.experimental.pallas import tpu_sc as plsc

B = 16384
R_IN = 7
R_OUT = 6
D = 128
LANES = 16

_info = plsc.get_sparse_core_info()
NC, NS = _info.num_cores, _info.num_subcores
NW = NC * NS
PER_W = B // NW
NCHUNK = 32
NSTEPS = PER_W // NCHUNK


def _body(xt_hbm, out_hbm, in0, in1, out0, out1, si0, si1, so0, so1):
    wid = lax.axis_index("s") * NC + lax.axis_index("c")
    base = wid * PER_W
    in_bufs = (in0, in1)
    out_bufs = (out0, out1)
    in_sems = (si0, si1)
    out_sems = (so0, so1)

    def start_in(step, par):
        off = base + step * NCHUNK
        return pltpu.async_copy(
            xt_hbm.at[pl.ds(0, R_IN), pl.ds(off, NCHUNK)],
            in_bufs[par], in_sems[par])

    def wait_in(par):
        pltpu.make_async_copy(
            xt_hbm.at[pl.ds(0, R_IN), pl.ds(0, NCHUNK)],
            in_bufs[par], in_sems[par]).wait()

    def start_out(step, par):
        off = base + step * NCHUNK
        return pltpu.async_copy(
            out_bufs[par], out_hbm.at[pl.ds(off, NCHUNK)], out_sems[par])

    def wait_out(par):
        pltpu.make_async_copy(
            out_bufs[par], out_hbm.at[pl.ds(0, NCHUNK)], out_sems[par]).wait()

    def run_compute(par):
        in_buf, out_buf = in_bufs[par], out_bufs[par]

        @plsc.parallel_loop(0, NCHUNK, unroll=4)
        def compute_one(i):
            for k in range(R_OUT):
                for v in range(D // LANES):
                    sl = pl.ds(v * LANES, LANES)
                    out_buf[i, k // 2, k % 2, sl] = (
                        in_buf[k, i, sl] + in_buf[k + 1, i, sl])

    start_in(0, 0)

    def pair_body(g, carry):
        s0 = 2 * g
        start_in(s0 + 1, 1)
        wait_in(0)

        @pl.when(g > 0)
        def _():
            wait_out(0)
        run_compute(0)
        start_out(s0, 0)

        @pl.when(s0 + 2 < NSTEPS)
        def _():
            start_in(s0 + 2, 0)
        wait_in(1)

        @pl.when(g > 0)
        def _():
            wait_out(1)
        run_compute(1)
        start_out(s0 + 1, 1)
        return carry

    lax.fori_loop(0, NSTEPS // 2, pair_body, 0)
    wait_out(0)
    wait_out(1)


def kernel(x):
    xt = jnp.transpose(x, (1, 0, 2))
    mesh = plsc.VectorSubcoreMesh(core_axis_name="c", subcore_axis_name="s")
    run = functools.partial(
        pl.kernel,
        mesh=mesh,
        out_type=jax.ShapeDtypeStruct((B, 3, 2, D), jnp.float32),
        compiler_params=pltpu.CompilerParams(use_tc_tiling_on_sc=True),
        scratch_types=[
            pltpu.VMEM((R_IN, NCHUNK, D), jnp.float32),
            pltpu.VMEM((R_IN, NCHUNK, D), jnp.float32),
            pltpu.VMEM((NCHUNK, 3, 2, D), jnp.float32),
            pltpu.VMEM((NCHUNK, 3, 2, D), jnp.float32),
            pltpu.SemaphoreType.DMA,
            pltpu.SemaphoreType.DMA,
            pltpu.SemaphoreType.DMA,
            pltpu.SemaphoreType.DMA,
        ],
    )(_body)
    return run(xt)
